# baseline (speedup 1.0000x reference)
import functools

import jax
import jax.numpy as jnp
from jax import lax
from jax.experimental import pallas as pl
from jax.experimental.pallas import tpu as pltpu
from jax.experimental.pallas import tpu_sc as plsc

EMBED_DIM = 64
HIDDEN = 128

NC = 2
NS = 16
NW = NC * NS
CHUNK = 128
K = 8


def _gather_body(table_hbm, idx_hbm, emb_hbm, idx_v, rows_v, gsem,
                 *, chunks_per_w, rows_per_w):
    wid = lax.axis_index("s") * NC + lax.axis_index("c")
    pltpu.sync_copy(idx_hbm.at[wid], idx_v)
    base = wid * rows_per_w
    n_groups = chunks_per_w // K

    def group(g, carry):
        copies = []
        for b in range(K):
            j = g * K + b
            copies.append(pltpu.async_copy(
                table_hbm.at[idx_v.at[j]],
                rows_v.at[pl.ds(b * CHUNK, CHUNK)],
                gsem))
        for c in copies:
            c.wait()
        pltpu.sync_copy(rows_v,
                        emb_hbm.at[pl.ds(base + g * (K * CHUNK), K * CHUNK)])
        return carry

    lax.fori_loop(0, n_groups, group, 0)


def _sc_gather(table, idx3d, m):
    chunks_per_w = idx3d.shape[1]
    rows_per_w = chunks_per_w * CHUNK
    mesh = plsc.VectorSubcoreMesh(core_axis_name="c", subcore_axis_name="s")
    body = functools.partial(_gather_body, chunks_per_w=chunks_per_w,
                             rows_per_w=rows_per_w)
    return pl.kernel(
        body,
        mesh=mesh,
        compiler_params=pltpu.CompilerParams(use_tc_tiling_on_sc=False),
        out_type=jax.ShapeDtypeStruct((m, EMBED_DIM), jnp.float32),
        scratch_types=[
            pltpu.VMEM((chunks_per_w, CHUNK), jnp.int32),
            pltpu.VMEM((K * CHUNK, EMBED_DIM), jnp.float32),
            pltpu.SemaphoreType.DMA,
        ],
    )(table, idx3d)


def _mm_body(e_ref, w_ref, o_ref):
    o_ref[...] = jnp.dot(e_ref[...], w_ref[...],
                         preferred_element_type=jnp.float32)


def _tc_project(emb, wt, m):
    bm = 4096
    return pl.pallas_call(
        _mm_body,
        grid=(m // bm,),
        in_specs=[
            pl.BlockSpec((bm, EMBED_DIM), lambda i: (i, 0)),
            pl.BlockSpec((EMBED_DIM, HIDDEN), lambda i: (0, 0)),
        ],
        out_specs=pl.BlockSpec((bm, HIDDEN), lambda i: (i, 0)),
        out_shape=jax.ShapeDtypeStruct((m, HIDDEN), jnp.float32),
    )(emb, wt)


def kernel(x, table, W):
    b, l = x.shape
    m = b * l
    chunks_per_w = m // (NW * CHUNK)
    idx3d = x.reshape(NW, chunks_per_w, CHUNK)
    emb = _sc_gather(table, idx3d, m)
    out = _tc_project(emb, W.T, m)
    return out.reshape(b, l, HIDDEN)


# speedup vs baseline: 1.2026x; 1.2026x over previous
import functools

import jax
import jax.numpy as jnp
from jax import lax
from jax.experimental import pallas as pl
from jax.experimental.pallas import tpu as pltpu
from jax.experimental.pallas import tpu_sc as plsc

EMBED_DIM = 64
HIDDEN = 128

NC = 2
NS = 16
NW = NC * NS
CHUNK = 128
K = 8


GROUP = K * CHUNK
BM = 2 * GROUP


def _gather_body(table_hbm, idx_hbm, emb2_hbm, idx_v, rows_v, gsem,
                 *, chunks_per_w):
    wid = lax.axis_index("s") * NC + lax.axis_index("c")
    pltpu.sync_copy(idx_hbm.at[wid], idx_v)
    n_groups = chunks_per_w // K
    gbase = wid * n_groups

    def group(g, carry):
        copies = []
        for b in range(K):
            j = g * K + b
            copies.append(pltpu.async_copy(
                table_hbm.at[idx_v.at[j]],
                rows_v.at[pl.ds(b * CHUNK, CHUNK)],
                gsem))
        for c in copies:
            c.wait()
        gg = gbase + g
        dst = emb2_hbm.at[pl.ds((gg // 2) * GROUP, GROUP),
                          pl.ds((gg % 2) * EMBED_DIM, EMBED_DIM)]
        pltpu.sync_copy(rows_v, dst)
        return carry

    lax.fori_loop(0, n_groups, group, 0)


def _sc_gather(table, idx3d, m):
    chunks_per_w = idx3d.shape[1]
    mesh = plsc.VectorSubcoreMesh(core_axis_name="c", subcore_axis_name="s")
    body = functools.partial(_gather_body, chunks_per_w=chunks_per_w)
    return pl.kernel(
        body,
        mesh=mesh,
        compiler_params=pltpu.CompilerParams(use_tc_tiling_on_sc=False),
        out_type=jax.ShapeDtypeStruct((m // 2, 2 * EMBED_DIM), jnp.float32),
        scratch_types=[
            pltpu.VMEM((chunks_per_w, CHUNK), jnp.int32),
            pltpu.VMEM((GROUP, EMBED_DIM), jnp.float32),
            pltpu.SemaphoreType.DMA,
        ],
    )(table, idx3d)


def _mm_body(e_ref, w_ref, o_ref):
    w = w_ref[...]
    e = e_ref[...]
    o_ref[pl.ds(0, GROUP), :] = jnp.dot(
        e[:, :EMBED_DIM], w, preferred_element_type=jnp.float32)
    o_ref[pl.ds(GROUP, GROUP), :] = jnp.dot(
        e[:, EMBED_DIM:], w, preferred_element_type=jnp.float32)


def _tc_project(emb2, wt, m):
    return pl.pallas_call(
        _mm_body,
        grid=(m // BM,),
        in_specs=[
            pl.BlockSpec((GROUP, 2 * EMBED_DIM), lambda i: (i, 0)),
            pl.BlockSpec((EMBED_DIM, HIDDEN), lambda i: (0, 0)),
        ],
        out_specs=pl.BlockSpec((BM, HIDDEN), lambda i: (i, 0)),
        out_shape=jax.ShapeDtypeStruct((m, HIDDEN), jnp.float32),
    )(emb2, wt)


def kernel(x, table, W):
    b, l = x.shape
    m = b * l
    chunks_per_w = m // (NW * CHUNK)
    idx3d = x.reshape(NW, chunks_per_w, CHUNK)
    emb2 = _sc_gather(table, idx3d, m)
    out = _tc_project(emb2, W.T, m)
    return out.reshape(b, l, HIDDEN)


# speedup vs baseline: 1.4151x; 1.1767x over previous
import functools

import jax
import jax.numpy as jnp
from jax import lax
from jax.experimental import pallas as pl
from jax.experimental.pallas import tpu as pltpu
from jax.experimental.pallas import tpu_sc as plsc

EMBED_DIM = 64
HIDDEN = 128

NC = 2
NS = 16
NW = NC * NS
CHUNK = 128
K = 8


GROUP = K * CHUNK
BM = 2 * GROUP


def _gather_body(table_hbm, idx_hbm, emb2_hbm, idx_v, rows_v, gsem,
                 *, chunks_per_w):
    wid = lax.axis_index("s") * NC + lax.axis_index("c")
    pltpu.sync_copy(idx_hbm.at[wid], idx_v)
    n_groups = chunks_per_w // K
    gbase = wid * n_groups

    def group(g, carry):
        copies = []
        for b in range(K):
            j = g * K + b
            copies.append(pltpu.async_copy(
                table_hbm.at[idx_v.at[j]],
                rows_v.at[pl.ds(b * CHUNK, CHUNK)],
                gsem))
        for c in copies:
            c.wait()
        gg = gbase + g
        dst = emb2_hbm.at[pl.ds((gg // 2) * GROUP, GROUP),
                          pl.ds((gg % 2) * EMBED_DIM, EMBED_DIM)]
        pltpu.sync_copy(rows_v, dst)
        return carry

    lax.fori_loop(0, n_groups, group, 0)


def _sc_gather(table, idx3d, m):
    chunks_per_w = idx3d.shape[1]
    mesh = plsc.VectorSubcoreMesh(core_axis_name="c", subcore_axis_name="s")
    body = functools.partial(_gather_body, chunks_per_w=chunks_per_w)
    return pl.kernel(
        body,
        mesh=mesh,
        compiler_params=pltpu.CompilerParams(use_tc_tiling_on_sc=False),
        out_type=jax.ShapeDtypeStruct((m // 2, 2 * EMBED_DIM), jnp.float32),
        scratch_types=[
            pltpu.VMEM((chunks_per_w, CHUNK), jnp.int32),
            pltpu.VMEM((GROUP, EMBED_DIM), jnp.float32),
            pltpu.SemaphoreType.DMA,
        ],
    )(table, idx3d)


TC_BM = 8192


def _mm_body(e_ref, w_ref, o_ref):
    w = w_ref[...]
    for p in range(TC_BM // BM):
        e = e_ref[pl.ds(p * GROUP, GROUP), :]
        o_ref[pl.ds(p * BM, GROUP), :] = jnp.dot(
            e[:, :EMBED_DIM], w, preferred_element_type=jnp.float32)
        o_ref[pl.ds(p * BM + GROUP, GROUP), :] = jnp.dot(
            e[:, EMBED_DIM:], w, preferred_element_type=jnp.float32)


def _tc_project(emb2, wt, m):
    return pl.pallas_call(
        _mm_body,
        grid=(m // TC_BM,),
        in_specs=[
            pl.BlockSpec((TC_BM // 2, 2 * EMBED_DIM), lambda i: (i, 0)),
            pl.BlockSpec((EMBED_DIM, HIDDEN), lambda i: (0, 0)),
        ],
        out_specs=pl.BlockSpec((TC_BM, HIDDEN), lambda i: (i, 0)),
        out_shape=jax.ShapeDtypeStruct((m, HIDDEN), jnp.float32),
    )(emb2, wt)


def kernel(x, table, W):
    b, l = x.shape
    m = b * l
    chunks_per_w = m // (NW * CHUNK)
    idx3d = x.reshape(NW, chunks_per_w, CHUNK)
    emb2 = _sc_gather(table, idx3d, m)
    out = _tc_project(emb2, W.T, m)
    return out.reshape(b, l, HIDDEN)


# speedup vs baseline: 1.4293x; 1.0100x over previous
import functools

import jax
import jax.numpy as jnp
from jax import lax
from jax.experimental import pallas as pl
from jax.experimental.pallas import tpu as pltpu
from jax.experimental.pallas import tpu_sc as plsc

EMBED_DIM = 64
HIDDEN = 128

NC = 2
NS = 16
NW = NC * NS
CHUNK = 128
K = 8


GROUP = K * CHUNK
BM = 2 * GROUP


def _gather_body(table_hbm, idx_hbm, emb2_hbm, idx_v, rows_v, gsem,
                 *, chunks_per_w):
    wid = lax.axis_index("s") * NC + lax.axis_index("c")
    pltpu.sync_copy(idx_hbm.at[wid], idx_v)
    n_groups = chunks_per_w // K
    gbase = wid * n_groups

    def group(g, carry):
        copies = []
        for b in range(K):
            j = g * K + b
            copies.append(pltpu.async_copy(
                table_hbm.at[idx_v.at[j]],
                rows_v.at[pl.ds(b * CHUNK, CHUNK)],
                gsem))
        for c in copies:
            c.wait()
        gg = gbase + g
        dst = emb2_hbm.at[pl.ds((gg // 2) * GROUP, GROUP),
                          pl.ds((gg % 2) * EMBED_DIM, EMBED_DIM)]
        pltpu.sync_copy(rows_v, dst)
        return carry

    lax.fori_loop(0, n_groups, group, 0)


def _sc_gather(table, idx3d, m):
    chunks_per_w = idx3d.shape[1]
    mesh = plsc.VectorSubcoreMesh(core_axis_name="c", subcore_axis_name="s")
    body = functools.partial(_gather_body, chunks_per_w=chunks_per_w)
    return pl.kernel(
        body,
        mesh=mesh,
        compiler_params=pltpu.CompilerParams(use_tc_tiling_on_sc=False),
        out_type=jax.ShapeDtypeStruct((m // 2, 2 * EMBED_DIM), jnp.float32),
        scratch_types=[
            pltpu.VMEM((chunks_per_w, CHUNK), jnp.int32),
            pltpu.VMEM((GROUP, EMBED_DIM), jnp.float32),
            pltpu.SemaphoreType.DMA,
        ],
    )(table, idx3d)


TC_BM = 16384


def _mm_body(e_ref, w_ref, o_ref):
    w = w_ref[...]
    for p in range(TC_BM // BM):
        e = e_ref[pl.ds(p * GROUP, GROUP), :]
        o_ref[pl.ds(p * BM, GROUP), :] = jnp.dot(
            e[:, :EMBED_DIM], w, preferred_element_type=jnp.float32)
        o_ref[pl.ds(p * BM + GROUP, GROUP), :] = jnp.dot(
            e[:, EMBED_DIM:], w, preferred_element_type=jnp.float32)


def _tc_project(emb2, wt, m):
    return pl.pallas_call(
        _mm_body,
        grid=(m // TC_BM,),
        in_specs=[
            pl.BlockSpec((TC_BM // 2, 2 * EMBED_DIM), lambda i: (i, 0)),
            pl.BlockSpec((EMBED_DIM, HIDDEN), lambda i: (0, 0)),
        ],
        out_specs=pl.BlockSpec((TC_BM, HIDDEN), lambda i: (i, 0)),
        out_shape=jax.ShapeDtypeStruct((m, HIDDEN), jnp.float32),
    )(emb2, wt)


def kernel(x, table, W):
    b, l = x.shape
    m = b * l
    chunks_per_w = m // (NW * CHUNK)
    idx3d = x.reshape(NW, chunks_per_w, CHUNK)
    emb2 = _sc_gather(table, idx3d, m)
    out = _tc_project(emb2, W.T, m)
    return out.reshape(b, l, HIDDEN)


# speedup vs baseline: 1.4406x; 1.0079x over previous
import functools

import jax
import jax.numpy as jnp
from jax import lax
from jax.experimental import pallas as pl
from jax.experimental.pallas import tpu as pltpu
from jax.experimental.pallas import tpu_sc as plsc

EMBED_DIM = 64
HIDDEN = 128

NC = 2
NS = 16
NW = NC * NS
CHUNK = 128
K = 8


GROUP = K * CHUNK
BM = 2 * GROUP


def _gather_body(table_hbm, idx_hbm, emb2_hbm, idx_v, rows_v, gsem,
                 *, chunks_per_w):
    wid = lax.axis_index("s") * NC + lax.axis_index("c")
    pltpu.sync_copy(idx_hbm.at[wid], idx_v)
    n_groups = chunks_per_w // K
    gbase = wid * n_groups

    def group(g, carry):
        copies = []
        for b in range(K):
            j = g * K + b
            copies.append(pltpu.async_copy(
                table_hbm.at[idx_v.at[j]],
                rows_v.at[pl.ds(b * CHUNK, CHUNK)],
                gsem))
        for c in copies:
            c.wait()
        gg = gbase + g
        dst = emb2_hbm.at[pl.ds((gg // 2) * GROUP, GROUP),
                          pl.ds((gg % 2) * EMBED_DIM, EMBED_DIM)]
        pltpu.sync_copy(rows_v, dst)
        return carry

    lax.fori_loop(0, n_groups, group, 0)


def _sc_gather(table, idx3d, m):
    chunks_per_w = idx3d.shape[1]
    mesh = plsc.VectorSubcoreMesh(core_axis_name="c", subcore_axis_name="s")
    body = functools.partial(_gather_body, chunks_per_w=chunks_per_w)
    return pl.kernel(
        body,
        mesh=mesh,
        compiler_params=pltpu.CompilerParams(use_tc_tiling_on_sc=False),
        out_type=jax.ShapeDtypeStruct((m // 2, 2 * EMBED_DIM), jnp.float32),
        scratch_types=[
            pltpu.VMEM((chunks_per_w, CHUNK), jnp.int32),
            pltpu.VMEM((GROUP, EMBED_DIM), jnp.float32),
            pltpu.SemaphoreType.DMA,
        ],
    )(table, idx3d)


TC_BM = 16384


def _mm_body(e_ref, w_ref, o_ref):
    w = w_ref[...]
    for p in range(TC_BM // BM):
        e = e_ref[pl.ds(p * GROUP, GROUP), :]
        o_ref[pl.ds(p * BM, GROUP), :] = jnp.dot(
            e[:, :EMBED_DIM], w, preferred_element_type=jnp.float32)
        o_ref[pl.ds(p * BM + GROUP, GROUP), :] = jnp.dot(
            e[:, EMBED_DIM:], w, preferred_element_type=jnp.float32)


def _mm_body_alias(e_ref, w_ref, oprev_ref, o_ref):
    del oprev_ref
    _mm_body(e_ref, w_ref, o_ref)


def _tc_project_chunk(emb2_c, wt, out_prev, c, m_c, m):
    nblk = m_c // TC_BM
    e_spec = pl.BlockSpec((TC_BM // 2, 2 * EMBED_DIM), lambda i: (i, 0))
    w_spec = pl.BlockSpec((EMBED_DIM, HIDDEN), lambda i: (0, 0))
    o_spec = pl.BlockSpec((TC_BM, HIDDEN),
                          lambda i, c=c, nblk=nblk: (c * nblk + i, 0))
    out_shape = jax.ShapeDtypeStruct((m, HIDDEN), jnp.float32)
    if out_prev is None:
        return pl.pallas_call(
            _mm_body,
            grid=(nblk,),
            in_specs=[e_spec, w_spec],
            out_specs=o_spec,
            out_shape=out_shape,
        )(emb2_c, wt)
    return pl.pallas_call(
        _mm_body_alias,
        grid=(nblk,),
        in_specs=[e_spec, w_spec,
                  pl.BlockSpec(memory_space=pltpu.MemorySpace.HBM)],
        out_specs=o_spec,
        out_shape=out_shape,
        input_output_aliases={2: 0},
    )(emb2_c, wt, out_prev)


N_CHUNKS = 5


def kernel(x, table, W):
    b, l = x.shape
    m = b * l
    m_c = m // N_CHUNKS
    cpw = m_c // (NW * CHUNK)
    x_flat = x.reshape(-1)
    wt = W.T
    out = None
    for c in range(N_CHUNKS):
        idx3d = x_flat[c * m_c:(c + 1) * m_c].reshape(NW, cpw, CHUNK)
        emb2_c = _sc_gather(table, idx3d, m_c)
        out = _tc_project_chunk(emb2_c, wt, out, c, m_c, m)
    return out.reshape(b, l, HIDDEN)
